# Initial kernel scaffold; baseline (speedup 1.0000x reference)
import functools

import jax
import jax.numpy as jnp
from jax import lax
from jax.experimental import pallas as pl
from jax.experimental.pallas import tpu as pltpu
from jax.experimental.pallas import tpu_sc as plsc

B = 4096
LSEQ = 200
VOCAB = 1000
VPAD = 1024
DIM = 128
NOUT = 100

NC = 2
NS = 16
LANES = 16
NW = NC * NS
ROWS_PER_W = B // NW
G = LANES
NG = ROWS_PER_W // G
CNT_WORDS = G * VPAD


def _sc_histogram(ids_flat):
    mesh = plsc.VectorSubcoreMesh(
        core_axis_name="c", subcore_axis_name="s",
        num_cores=NC, num_subcores=NS)

    @functools.partial(
        pl.kernel,
        mesh=mesh,
        out_type=jax.ShapeDtypeStruct((B * VPAD,), jnp.float32),
        scratch_types=[
            pltpu.VMEM((G * LSEQ,), jnp.int32),
            pltpu.VMEM((CNT_WORDS,), jnp.float32),
        ],
    )
    def hist_kernel(ids_hbm, counts_hbm, ids_v, cnt_v):
        wid = lax.axis_index("s") * NC + lax.axis_index("c")
        iota = lax.iota(jnp.int32, LANES)
        row_base_ids = iota * LSEQ
        row_base_cnt = iota * VPAD
        ones = jnp.ones((LANES,), jnp.float32)
        zeros = jnp.zeros((LANES,), jnp.float32)

        def group(g, carry):
            base = wid * ROWS_PER_W + g * G
            pltpu.sync_copy(
                ids_hbm.at[pl.ds(base * LSEQ, G * LSEQ)], ids_v)

            def zero_body(j, c):
                for u in range(8):
                    cnt_v[pl.ds((j * 8 + u) * LANES, LANES)] = zeros
                return c
            lax.fori_loop(0, CNT_WORDS // (LANES * 8), zero_body, 0)

            def pos_body(j, c):
                for u in range(8):
                    l = j * 8 + u
                    idv = plsc.load_gather(ids_v, [row_base_ids + l])
                    plsc.addupdate_scatter(
                        cnt_v, [row_base_cnt + idv], ones)
                return c
            lax.fori_loop(0, LSEQ // 8, pos_body, 0)

            pltpu.sync_copy(
                cnt_v, counts_hbm.at[pl.ds(base * VPAD, CNT_WORDS)])
            return carry

        lax.fori_loop(0, NG, group, 0)

    return hist_kernel(ids_flat)


def _tc_body(cnt_ref, emb_ref, w_ref, b_ref, out_ref):
    m = jnp.dot(emb_ref[...], w_ref[...],
                preferred_element_type=jnp.float32,
                precision=lax.Precision.HIGHEST)
    acc = jnp.dot(cnt_ref[...], m,
                  preferred_element_type=jnp.float32,
                  precision=lax.Precision.HIGHEST)
    out_ref[...] = acc * (1.0 / LSEQ) + b_ref[...]


def _tc_logits(counts, emb_pad, w, b2d):
    grid = 8
    blk = B // grid
    return pl.pallas_call(
        _tc_body,
        grid=(grid,),
        in_specs=[
            pl.BlockSpec((blk, VPAD), lambda i: (i, 0)),
            pl.BlockSpec((VPAD, DIM), lambda i: (0, 0)),
            pl.BlockSpec((DIM, NOUT), lambda i: (0, 0)),
            pl.BlockSpec((1, NOUT), lambda i: (0, 0)),
        ],
        out_specs=pl.BlockSpec((blk, NOUT), lambda i: (i, 0)),
        out_shape=jax.ShapeDtypeStruct((B, NOUT), jnp.float32),
    )(counts, emb_pad, w, b2d)


def kernel(input_ids, embedding_table, W, b):
    ids_flat = input_ids.astype(jnp.int32).reshape(-1)
    counts = _sc_histogram(ids_flat).reshape(B, VPAD)
    emb_pad = jnp.pad(embedding_table, ((0, VPAD - VOCAB), (0, 0)))
    return _tc_logits(counts, emb_pad, W, b.reshape(1, NOUT))


# speedup vs baseline: 26.2945x; 26.2945x over previous
import functools

import jax
import jax.numpy as jnp
from jax import lax
from jax.experimental import pallas as pl
from jax.experimental.pallas import tpu as pltpu
from jax.experimental.pallas import tpu_sc as plsc

B = 4096
LSEQ = 200
VOCAB = 1000
VPAD = 1024
DIM = 128
NOUT = 100

NC = 2
NS = 16
LANES = 16
NW = NC * NS
ROWS_PER_W = B // NW
G = LANES
NG = ROWS_PER_W // G
CNT_WORDS = G * VPAD


def _sc_histogram(ids_flat):
    mesh = plsc.VectorSubcoreMesh(
        core_axis_name="c", subcore_axis_name="s",
        num_cores=NC, num_subcores=NS)

    @functools.partial(
        pl.kernel,
        mesh=mesh,
        compiler_params=pltpu.CompilerParams(needs_layout_passes=False),
        out_type=jax.ShapeDtypeStruct((B * VPAD,), jnp.float32),
        scratch_types=[
            pltpu.VMEM((G * LSEQ,), jnp.int32),
            pltpu.VMEM((CNT_WORDS,), jnp.float32),
        ],
    )
    def hist_kernel(ids_hbm, counts_hbm, ids_v, cnt_v):
        wid = lax.axis_index("s") * NC + lax.axis_index("c")
        iota = lax.iota(jnp.int32, LANES)
        row_base_ids = iota * LSEQ
        row_base_cnt = iota * VPAD
        ones = jnp.ones((LANES,), jnp.float32)
        zeros = jnp.zeros((LANES,), jnp.float32)

        def group(g, carry):
            base = wid * ROWS_PER_W + g * G
            pltpu.sync_copy(
                ids_hbm.at[pl.ds(base * LSEQ, G * LSEQ)], ids_v)

            def zero_body(j, c):
                for u in range(8):
                    cnt_v[pl.ds((j * 8 + u) * LANES, LANES)] = zeros
                return c
            lax.fori_loop(0, CNT_WORDS // (LANES * 8), zero_body, 0)

            def pos_body(j, c):
                for u in range(8):
                    l = j * 8 + u
                    idv = plsc.load_gather(ids_v, [row_base_ids + l])
                    plsc.addupdate_scatter(
                        cnt_v, [row_base_cnt + idv], ones)
                return c
            lax.fori_loop(0, LSEQ // 8, pos_body, 0)

            pltpu.sync_copy(
                cnt_v, counts_hbm.at[pl.ds(base * VPAD, CNT_WORDS)])
            return carry

        lax.fori_loop(0, NG, group, 0)

    return hist_kernel(ids_flat)


def _tc_body(cnt_ref, emb_ref, w_ref, b_ref, out_ref):
    m = jnp.dot(emb_ref[...], w_ref[...],
                preferred_element_type=jnp.float32,
                precision=lax.Precision.HIGHEST)
    acc = jnp.dot(cnt_ref[...], m,
                  preferred_element_type=jnp.float32,
                  precision=lax.Precision.HIGHEST)
    out_ref[...] = acc * (1.0 / LSEQ) + b_ref[...]


def _tc_logits(counts, emb_pad, w, b2d):
    grid = 8
    blk = B // grid
    return pl.pallas_call(
        _tc_body,
        grid=(grid,),
        in_specs=[
            pl.BlockSpec((blk, VPAD), lambda i: (i, 0)),
            pl.BlockSpec((VPAD, DIM), lambda i: (0, 0)),
            pl.BlockSpec((DIM, NOUT), lambda i: (0, 0)),
            pl.BlockSpec((1, NOUT), lambda i: (0, 0)),
        ],
        out_specs=pl.BlockSpec((blk, NOUT), lambda i: (i, 0)),
        out_shape=jax.ShapeDtypeStruct((B, NOUT), jnp.float32),
    )(counts, emb_pad, w, b2d)


def kernel(input_ids, embedding_table, W, b):
    ids_flat = input_ids.astype(jnp.int32).reshape(-1)
    counts = _sc_histogram(ids_flat).reshape(B, VPAD)
    emb_pad = jnp.pad(embedding_table, ((0, VPAD - VOCAB), (0, 0)))
    return _tc_logits(counts, emb_pad, W, b.reshape(1, NOUT))


# speedup vs baseline: 32.8944x; 1.2510x over previous
import functools

import jax
import jax.numpy as jnp
from jax import lax
from jax.experimental import pallas as pl
from jax.experimental.pallas import tpu as pltpu
from jax.experimental.pallas import tpu_sc as plsc

B = 4096
LSEQ = 200
VOCAB = 1000
VPAD = 1024
DIM = 128
NOUT = 100

NC = 2
NS = 16
LANES = 16
NW = NC * NS
ROWS_PER_W = B // NW
G = LANES
NG = ROWS_PER_W // G


def _sc_histogram(ids):
    mesh = plsc.VectorSubcoreMesh(
        core_axis_name="c", subcore_axis_name="s",
        num_cores=NC, num_subcores=NS)

    @functools.partial(
        pl.kernel,
        mesh=mesh,
        compiler_params=pltpu.CompilerParams(needs_layout_passes=False),
        out_type=jax.ShapeDtypeStruct((B, VPAD), jnp.float32),
        scratch_types=[
            pltpu.VMEM((G, LSEQ), jnp.int32),
            pltpu.VMEM((G, VPAD), jnp.float32),
        ],
    )
    def hist_kernel(ids_hbm, counts_hbm, ids_v, cnt_v):
        wid = lax.axis_index("s") * NC + lax.axis_index("c")
        iota = lax.iota(jnp.int32, LANES)
        ones = jnp.ones((LANES,), jnp.float32)
        zeros = jnp.zeros((LANES,), jnp.float32)

        def group(g, carry):
            base = wid * ROWS_PER_W + g * G
            pltpu.sync_copy(ids_hbm.at[pl.ds(base, G), :], ids_v)

            def zero_body(j, c):
                for r in range(G):
                    cnt_v[r, pl.ds(j * LANES, LANES)] = zeros
                return c
            lax.fori_loop(0, VPAD // LANES, zero_body, 0)

            def pos_body(j, c):
                for u in range(8):
                    l = j * 8 + u
                    col = jnp.full((LANES,), l, jnp.int32)
                    idv = plsc.load_gather(ids_v, [iota, col])
                    plsc.addupdate_scatter(cnt_v, [iota, idv], ones)
                return c
            lax.fori_loop(0, LSEQ // 8, pos_body, 0)

            pltpu.sync_copy(cnt_v, counts_hbm.at[pl.ds(base, G), :])
            return carry

        lax.fori_loop(0, NG, group, 0)

    return hist_kernel(ids)


def _tc_body(cnt_ref, emb_ref, w_ref, b_ref, out_ref):
    m = jnp.dot(emb_ref[...], w_ref[...],
                preferred_element_type=jnp.float32,
                precision=lax.Precision.HIGHEST)
    acc = jnp.dot(cnt_ref[...], m,
                  preferred_element_type=jnp.float32)
    out_ref[...] = acc * (1.0 / LSEQ) + b_ref[...]


def _tc_logits(counts, emb_pad, w, b2d):
    grid = 8
    blk = B // grid
    return pl.pallas_call(
        _tc_body,
        grid=(grid,),
        in_specs=[
            pl.BlockSpec((blk, VPAD), lambda i: (i, 0)),
            pl.BlockSpec((VPAD, DIM), lambda i: (0, 0)),
            pl.BlockSpec((DIM, NOUT), lambda i: (0, 0)),
            pl.BlockSpec((1, NOUT), lambda i: (0, 0)),
        ],
        out_specs=pl.BlockSpec((blk, NOUT), lambda i: (i, 0)),
        out_shape=jax.ShapeDtypeStruct((B, NOUT), jnp.float32),
    )(counts, emb_pad, w, b2d)


def kernel(input_ids, embedding_table, W, b):
    ids = input_ids.astype(jnp.int32)
    counts = _sc_histogram(ids)
    emb_pad = jnp.pad(embedding_table, ((0, VPAD - VOCAB), (0, 0)))
    return _tc_logits(counts, emb_pad, W, b.reshape(1, NOUT))
